# baseline (speedup 1.0000x reference)
import jax
import jax.numpy as jnp
from jax import lax
from jax.experimental import pallas as pl
from jax.experimental.pallas import tpu as pltpu
from jax.experimental.pallas import tpu_sc as plsc

NUM_CORES = 2
NUM_SUBCORES = 16
NUM_WORKERS = NUM_CORES * NUM_SUBCORES
LANES = 16

VOCAB = 1000000
EMBED_DIM = 64
BATCH = 16384
SLICE = BATCH // NUM_WORKERS
BLKW = 512
BLK_SH = 9
RND_SH = 5
VMAIN = (VOCAB // BLKW) * BLKW
NBT = VMAIN // BLKW
NBLOCKS = 2 * NBT
NROUNDS = -(-NBLOCKS // NUM_WORKERS)

ICHUNK = 4096
NICH = 3 * BATCH // ICHUNK
MYCAP = 8192
SINK = 3 * BATCH
N_INTER = (3 * BATCH + LANES) * EMBED_DIM
NRING = 8
ROW_BYTES = EMBED_DIM * 4


def _gather_body(in_tab, out_tab, tail_in, tail_out, idx_all,
                 inter,
                 idxc, blk_a, blk_b, tail_v, ring, my_idx, my_b,
                 exc_v, exb_v,
                 sem_a, sem_b, sem_w):
    cid = lax.axis_index("c")
    sid = lax.axis_index("s")
    wid = sid * NUM_CORES + cid
    iota16 = lax.iota(jnp.int32, LANES)

    pltpu.sync_copy(tail_in, tail_v.at[pl.ds(0, EMBED_DIM)])
    pltpu.sync_copy(tail_out, tail_v.at[pl.ds(EMBED_DIM, EMBED_DIM)])

    def wait_one_row():
        pltpu.make_async_copy(
            inter.at[pl.ds(0, EMBED_DIM)],
            ring.at[pl.ds(0, EMBED_DIM)], sem_w).wait()

    def ring_write(h, src_blk, d_base, c16, b):
        def wait_slot(c):
            wait_one_row()
            return c
        lax.cond(h >= NRING, wait_slot, lambda c: c, 0)
        slot = lax.rem(h, NRING) * EMBED_DIM
        for k in range(EMBED_DIM // LANES):
            ring[pl.ds(slot + k * LANES, LANES)] = plsc.load_gather(
                src_blk, [d_base + k * LANES + iota16, c16])
        pltpu.async_copy(
            ring.at[pl.ds(slot, EMBED_DIM)],
            inter.at[pl.ds(b * EMBED_DIM, EMBED_DIM)], sem_w)
        return h + 1

    def extract_lanes(h, m, src_blk, col, bp, toff):
        mi = m.astype(jnp.int32)
        for i in range(LANES):
            def do(hh):
                c16 = lax.broadcast(col[i], (LANES,))
                return ring_write(hh, src_blk, toff[i], c16, bp[i])
            h = lax.cond(mi[i] > 0, do, lambda hh: hh, h)
        return h

    def chunk_scan(ch, carry):
        pltpu.sync_copy(idx_all.at[pl.ds(ch * ICHUNK, ICHUNK)], idxc)

        def vec_scan(v, carry):
            h, off = carry
            idx = idxc[pl.ds(v * LANES, LANES)]
            bpos = ch * ICHUNK + v * LANES + iota16
            tj = jnp.where(bpos < BATCH, 0, NBT)
            j = tj + lax.shift_right_logical(idx, BLK_SH)
            m = (idx < VMAIN) & (lax.rem(j, NUM_WORKERS) == wid)
            npop = plsc.all_reduce_population_count(m)

            def emit(carry):
                h, off = carry
                offc = jnp.minimum(off, MYCAP - LANES)
                entry = lax.shift_left(j, BLK_SH + 1) | (idx & (BLKW - 1))
                plsc.store_compressed(
                    my_idx.at[pl.ds(offc, LANES)], entry, mask=m)
                plsc.store_compressed(
                    my_b.at[pl.ds(offc, LANES)], bpos, mask=m)
                return (h, off + npop[0])

            carry = lax.cond(npop[0] > 0, emit, lambda c: c, (h, off))

            mt = (idx >= VMAIN) & ((idx & (NUM_WORKERS - 1)) == wid)
            npt = plsc.all_reduce_population_count(mt)

            def emit_t(carry):
                h, off = carry
                row = jnp.where(mt, idx - VMAIN, 0)
                toff = jnp.where(bpos < BATCH, 0, EMBED_DIM)
                h = extract_lanes(h, mt, tail_v, row, bpos, toff)
                return (h, off)

            return lax.cond(npt[0] > 0, emit_t, lambda c: c, carry)

        return lax.fori_loop(0, ICHUNK // LANES, vec_scan, carry)

    h_count, nh = lax.fori_loop(0, NICH, chunk_scan, (0, 0))
    ovf = nh > MYCAP - LANES

    def load_block(r, dst, sem):
        j = r * NUM_WORKERS + wid

        @pl.when(j < NBT)
        def _():
            off = pl.multiple_of(j * BLKW, BLKW)
            pltpu.async_copy(in_tab.at[:, pl.ds(off, BLKW)], dst, sem)

        @pl.when((j >= NBT) & (j < NBLOCKS))
        def _():
            off = pl.multiple_of((j - NBT) * BLKW, BLKW)
            pltpu.async_copy(out_tab.at[:, pl.ds(off, BLKW)], dst, sem)

    def wait_block(r, dst, sem):
        j = r * NUM_WORKERS + wid

        @pl.when(j < NBLOCKS)
        def _():
            pltpu.make_async_copy(
                in_tab.at[:, pl.ds(0, BLKW)], dst, sem).wait()

    load_block(0, blk_a, sem_a)

    def make_round(src_blk, sem_cur, nxt_blk, sem_nxt):
        def round_body(r, h):
            load_block(r + 1, nxt_blk, sem_nxt)
            wait_block(r, src_blk, sem_cur)

            zero16 = jnp.zeros((LANES,), jnp.int32)
            sink16 = jnp.full((LANES,), SINK, jnp.int32)

            def fast(h):
                def sift(v, off2):
                    e = my_idx[pl.ds(v * LANES, LANES)]
                    bp = my_b[pl.ds(v * LANES, LANES)]
                    valid = (v * LANES + iota16) < nh
                    m = valid & (lax.shift_right_logical(
                        e, BLK_SH + 1 + RND_SH) == r)
                    npop = plsc.all_reduce_population_count(m)

                    def emit(off2):
                        plsc.store_compressed(
                            exc_v.at[pl.ds(off2, LANES)],
                            e & (BLKW - 1), mask=m)
                        plsc.store_compressed(
                            exb_v.at[pl.ds(off2, LANES)], bp, mask=m)
                        return off2 + npop[0]

                    return lax.cond(npop[0] > 0, emit, lambda o: o, off2)

                off2 = lax.fori_loop(
                    0, lax.div(nh + LANES - 1, LANES), sift, 0)
                exc_v[pl.ds(off2, LANES)] = zero16
                exb_v[pl.ds(off2, LANES)] = sink16

                def extract(v, h):
                    col = exc_v[pl.ds(v * LANES, LANES)]
                    bp = exb_v[pl.ds(v * LANES, LANES)]
                    for i in range(LANES):
                        c16 = lax.broadcast(col[i], (LANES,))
                        h = ring_write(h, src_blk, 0, c16, bp[i])
                    return h

                return lax.fori_loop(
                    0, lax.div(off2 + LANES - 1, LANES), extract, h)

            def slow(h):
                myj = r * NUM_WORKERS + wid

                def chunk(ch, h):
                    pltpu.sync_copy(
                        idx_all.at[pl.ds(ch * ICHUNK, ICHUNK)], idxc)

                    def vec(v, h):
                        idx = idxc[pl.ds(v * LANES, LANES)]
                        bpos = ch * ICHUNK + v * LANES + iota16
                        tj = jnp.where(bpos < BATCH, 0, NBT)
                        j = tj + lax.shift_right_logical(idx, BLK_SH)
                        m = (idx < VMAIN) & (j == myj)
                        npop = plsc.all_reduce_population_count(m)

                        def emit(h):
                            return extract_lanes(
                                h, m, src_blk, idx & (BLKW - 1), bpos,
                                zero16)

                        return lax.cond(npop[0] > 0, emit, lambda hh: hh, h)

                    return lax.fori_loop(0, ICHUNK // LANES, vec, h)

                return lax.fori_loop(0, NICH, chunk, h)

            return lax.cond(ovf, slow, fast, h)

        return round_body

    even = make_round(blk_a, sem_a, blk_b, sem_b)
    odd = make_round(blk_b, sem_b, blk_a, sem_a)

    def two_rounds(rr, h):
        h = even(2 * rr, h)
        h = odd(2 * rr + 1, h)
        return h

    h_count = lax.fori_loop(0, NROUNDS // 2, two_rounds, h_count)
    if NROUNDS % 2:
        h_count = even(NROUNDS - 1, h_count)

    def drain(i, c):
        wait_one_row()
        return c
    lax.fori_loop(0, jnp.minimum(h_count, NRING), drain, 0)


def _score_body(inter, pos_out, neg_out,
                rows_v, pacc_v, nacc_v, score_pos, score_neg, sem):
    wid = lax.axis_index("s") * NUM_CORES + lax.axis_index("c")
    base = wid * SLICE
    iota16 = lax.iota(jnp.int32, LANES)

    copies = []
    for t in range(3):
        copies.append(pltpu.async_copy(
            inter.at[pl.ds((t * BATCH + base) * EMBED_DIM,
                           SLICE * EMBED_DIM)],
            rows_v.at[pl.ds(t * SLICE * EMBED_DIM, SLICE * EMBED_DIM)],
            sem))
    for c in copies:
        c.wait()

    def chunk_body(c, carry):
        for i in range(LANES):
            r = c * LANES + i
            accp = jnp.zeros((LANES,), jnp.float32)
            accn = jnp.zeros((LANES,), jnp.float32)
            for k in range(EMBED_DIM // LANES):
                o = r * EMBED_DIM + k * LANES
                a = rows_v[pl.ds(o, LANES)]
                p = rows_v[pl.ds(SLICE * EMBED_DIM + o, LANES)]
                n = rows_v[pl.ds(2 * SLICE * EMBED_DIM + o, LANES)]
                accp = accp + a * p
                accn = accn + a * n
            pacc_v[pl.ds(i * LANES, LANES)] = accp
            nacc_v[pl.ds(i * LANES, LANES)] = accn
        totp = jnp.zeros((LANES,), jnp.float32)
        totn = jnp.zeros((LANES,), jnp.float32)
        for d in range(LANES):
            gidx = iota16 * LANES + d
            totp = totp + plsc.load_gather(pacc_v, [gidx])
            totn = totn + plsc.load_gather(nacc_v, [gidx])
        score_pos[pl.ds(c * LANES, LANES)] = totp
        score_neg[pl.ds(c * LANES, LANES)] = totn
        return carry

    lax.fori_loop(0, SLICE // LANES, chunk_body, 0)

    pltpu.sync_copy(score_pos, pos_out.at[pl.ds(base, SLICE)])
    pltpu.sync_copy(score_neg, neg_out.at[pl.ds(base, SLICE)])


def _mesh():
    return plsc.VectorSubcoreMesh(
        core_axis_name="c", subcore_axis_name="s",
        num_cores=NUM_CORES, num_subcores=NUM_SUBCORES)


@jax.jit
def _skipgram_scores(in_tab, out_tab, tail_in, tail_out, idx_all):
    params = pltpu.CompilerParams(
        needs_layout_passes=False, use_tc_tiling_on_sc=True)
    gather = pl.kernel(
        _gather_body,
        out_type=jax.ShapeDtypeStruct((N_INTER,), jnp.float32),
        mesh=_mesh(),
        scratch_types=[
            pltpu.VMEM((ICHUNK,), jnp.int32),
            pltpu.VMEM((EMBED_DIM, BLKW), jnp.float32),
            pltpu.VMEM((EMBED_DIM, BLKW), jnp.float32),
            pltpu.VMEM((2 * EMBED_DIM, EMBED_DIM), jnp.float32),
            pltpu.VMEM((NRING * EMBED_DIM,), jnp.float32),
            pltpu.VMEM((MYCAP,), jnp.int32),
            pltpu.VMEM((MYCAP,), jnp.int32),
            pltpu.VMEM((MYCAP + LANES,), jnp.int32),
            pltpu.VMEM((MYCAP + LANES,), jnp.int32),
            pltpu.SemaphoreType.DMA,
            pltpu.SemaphoreType.DMA,
            pltpu.SemaphoreType.DMA,
        ],
        compiler_params=params,
    )
    inter = gather(in_tab, out_tab, tail_in, tail_out, idx_all)
    score = pl.kernel(
        _score_body,
        out_type=(
            jax.ShapeDtypeStruct((BATCH,), jnp.float32),
            jax.ShapeDtypeStruct((BATCH,), jnp.float32),
        ),
        mesh=_mesh(),
        scratch_types=[
            pltpu.VMEM((3 * SLICE * EMBED_DIM,), jnp.float32),
            pltpu.VMEM((LANES * LANES,), jnp.float32),
            pltpu.VMEM((LANES * LANES,), jnp.float32),
            pltpu.VMEM((SLICE,), jnp.float32),
            pltpu.VMEM((SLICE,), jnp.float32),
            pltpu.SemaphoreType.DMA,
        ],
        compiler_params=params,
    )
    return score(inter)


def kernel(input_labels, pos_labels, neg_labels, in_embed, out_embed):
    in_tab = in_embed.T
    out_tab = out_embed.T
    tail_in = in_embed[VMAIN:, :].T
    tail_out = out_embed[VMAIN:, :].T
    idx_all = jnp.concatenate([
        input_labels.astype(jnp.int32),
        pos_labels.astype(jnp.int32),
        neg_labels.astype(jnp.int32)])
    pos_score, neg_score = _skipgram_scores(
        in_tab, out_tab, tail_in, tail_out, idx_all)
    return pos_score, neg_score.reshape(BATCH, 1)


# speedup vs baseline: 1.5198x; 1.5198x over previous
import jax
import jax.numpy as jnp
from jax import lax
from jax.experimental import pallas as pl
from jax.experimental.pallas import tpu as pltpu
from jax.experimental.pallas import tpu_sc as plsc

NUM_CORES = 2
NUM_SUBCORES = 16
NUM_WORKERS = NUM_CORES * NUM_SUBCORES
LANES = 16

VOCAB = 1000000
EMBED_DIM = 64
BATCH = 16384
SLICE = BATCH // NUM_WORKERS
BLKW = 512
BLK_SH = 9
RND_SH = 5
VMAIN = (VOCAB // BLKW) * BLKW
NBT = VMAIN // BLKW
NBLOCKS = 2 * NBT
NROUNDS = -(-NBLOCKS // NUM_WORKERS)

ICHUNK = 4096
NICH = 3 * BATCH // ICHUNK
MYCAP = 8192
SINK = 3 * BATCH
N_INTER = (3 * BATCH + LANES) * EMBED_DIM
NRING = 8
ROW_BYTES = EMBED_DIM * 4


def _gather_body(in_tab, out_tab, tail_in, tail_out, idx_all,
                 inter,
                 idxc, blk_a, blk_b, tail_v, ring, my_idx, my_b,
                 sem_a, sem_b, sem_w):
    cid = lax.axis_index("c")
    sid = lax.axis_index("s")
    wid = sid * NUM_CORES + cid
    iota16 = lax.iota(jnp.int32, LANES)

    pltpu.sync_copy(tail_in, tail_v.at[pl.ds(0, EMBED_DIM)])
    pltpu.sync_copy(tail_out, tail_v.at[pl.ds(EMBED_DIM, EMBED_DIM)])

    def wait_one_row():
        pltpu.make_async_copy(
            inter.at[pl.ds(0, EMBED_DIM)],
            ring.at[pl.ds(0, EMBED_DIM)], sem_w).wait()

    def ring_write(h, src_blk, d_base, c16, b):
        def wait_slot(c):
            wait_one_row()
            return c
        lax.cond(h >= NRING, wait_slot, lambda c: c, 0)
        slot = lax.rem(h, NRING) * EMBED_DIM
        for k in range(EMBED_DIM // LANES):
            ring[pl.ds(slot + k * LANES, LANES)] = plsc.load_gather(
                src_blk, [d_base + k * LANES + iota16, c16])
        pltpu.async_copy(
            ring.at[pl.ds(slot, EMBED_DIM)],
            inter.at[pl.ds(b * EMBED_DIM, EMBED_DIM)], sem_w)
        return h + 1

    def extract_lanes(h, m, src_blk, col, bp, toff):
        mi = m.astype(jnp.int32)
        for i in range(LANES):
            def do(hh):
                c16 = lax.broadcast(col[i], (LANES,))
                return ring_write(hh, src_blk, toff[i], c16, bp[i])
            h = lax.cond(mi[i] > 0, do, lambda hh: hh, h)
        return h

    def chunk_scan(ch, carry):
        pltpu.sync_copy(idx_all.at[pl.ds(ch * ICHUNK, ICHUNK)], idxc)

        def vec_scan(v, carry):
            h, off = carry
            idx = idxc[pl.ds(v * LANES, LANES)]
            bpos = ch * ICHUNK + v * LANES + iota16
            tj = jnp.where(bpos < BATCH, 0, NBT)
            j = tj + lax.shift_right_logical(idx, BLK_SH)
            m = (idx < VMAIN) & (lax.rem(j, NUM_WORKERS) == wid)
            npop = plsc.all_reduce_population_count(m)

            def emit(carry):
                h, off = carry
                offc = jnp.minimum(off, MYCAP - LANES)
                entry = lax.shift_left(j, BLK_SH + 1) | (idx & (BLKW - 1))
                plsc.store_compressed(
                    my_idx.at[pl.ds(offc, LANES)], entry, mask=m)
                plsc.store_compressed(
                    my_b.at[pl.ds(offc, LANES)], bpos, mask=m)
                return (h, off + npop[0])

            carry = lax.cond(npop[0] > 0, emit, lambda c: c, (h, off))

            mt = (idx >= VMAIN) & ((idx & (NUM_WORKERS - 1)) == wid)
            npt = plsc.all_reduce_population_count(mt)

            def emit_t(carry):
                h, off = carry
                row = jnp.where(mt, idx - VMAIN, 0)
                toff = jnp.where(bpos < BATCH, 0, EMBED_DIM)
                h = extract_lanes(h, mt, tail_v, row, bpos, toff)
                return (h, off)

            return lax.cond(npt[0] > 0, emit_t, lambda c: c, carry)

        return lax.fori_loop(0, ICHUNK // LANES, vec_scan, carry)

    h_count, nh = lax.fori_loop(0, NICH, chunk_scan, (0, 0))
    ovf = nh > MYCAP - LANES

    def load_block(r, dst, sem):
        j = r * NUM_WORKERS + wid

        @pl.when(j < NBT)
        def _():
            off = pl.multiple_of(j * BLKW, BLKW)
            pltpu.async_copy(in_tab.at[:, pl.ds(off, BLKW)], dst, sem)

        @pl.when((j >= NBT) & (j < NBLOCKS))
        def _():
            off = pl.multiple_of((j - NBT) * BLKW, BLKW)
            pltpu.async_copy(out_tab.at[:, pl.ds(off, BLKW)], dst, sem)

    def wait_block(r, dst, sem):
        j = r * NUM_WORKERS + wid

        @pl.when(j < NBLOCKS)
        def _():
            pltpu.make_async_copy(
                in_tab.at[:, pl.ds(0, BLKW)], dst, sem).wait()

    load_block(0, blk_a, sem_a)

    def make_round(src_blk, sem_cur, nxt_blk, sem_nxt):
        def round_body(r, h):
            load_block(r + 1, nxt_blk, sem_nxt)
            wait_block(r, src_blk, sem_cur)

            zero16 = jnp.zeros((LANES,), jnp.int32)

            def fast(h):
                def sift(v, h):
                    e = my_idx[pl.ds(v * LANES, LANES)]
                    bp = my_b[pl.ds(v * LANES, LANES)]
                    valid = (v * LANES + iota16) < nh
                    m = valid & (lax.shift_right_logical(
                        e, BLK_SH + 1 + RND_SH) == r)
                    npop = plsc.all_reduce_population_count(m)

                    def emit(h):
                        return extract_lanes(
                            h, m, src_blk, e & (BLKW - 1), bp, zero16)

                    return lax.cond(npop[0] > 0, emit, lambda hh: hh, h)

                return lax.fori_loop(
                    0, lax.div(nh + LANES - 1, LANES), sift, h)

            def slow(h):
                myj = r * NUM_WORKERS + wid

                def chunk(ch, h):
                    pltpu.sync_copy(
                        idx_all.at[pl.ds(ch * ICHUNK, ICHUNK)], idxc)

                    def vec(v, h):
                        idx = idxc[pl.ds(v * LANES, LANES)]
                        bpos = ch * ICHUNK + v * LANES + iota16
                        tj = jnp.where(bpos < BATCH, 0, NBT)
                        j = tj + lax.shift_right_logical(idx, BLK_SH)
                        m = (idx < VMAIN) & (j == myj)
                        npop = plsc.all_reduce_population_count(m)

                        def emit(h):
                            return extract_lanes(
                                h, m, src_blk, idx & (BLKW - 1), bpos,
                                zero16)

                        return lax.cond(npop[0] > 0, emit, lambda hh: hh, h)

                    return lax.fori_loop(0, ICHUNK // LANES, vec, h)

                return lax.fori_loop(0, NICH, chunk, h)

            return lax.cond(ovf, slow, fast, h)

        return round_body

    even = make_round(blk_a, sem_a, blk_b, sem_b)
    odd = make_round(blk_b, sem_b, blk_a, sem_a)

    def two_rounds(rr, h):
        h = even(2 * rr, h)
        h = odd(2 * rr + 1, h)
        return h

    h_count = lax.fori_loop(0, NROUNDS // 2, two_rounds, h_count)
    if NROUNDS % 2:
        h_count = even(NROUNDS - 1, h_count)

    def drain(i, c):
        wait_one_row()
        return c
    lax.fori_loop(0, jnp.minimum(h_count, NRING), drain, 0)


def _score_body(inter, pos_out, neg_out,
                rows_v, pacc_v, nacc_v, score_pos, score_neg, sem):
    wid = lax.axis_index("s") * NUM_CORES + lax.axis_index("c")
    base = wid * SLICE
    iota16 = lax.iota(jnp.int32, LANES)

    copies = []
    for t in range(3):
        copies.append(pltpu.async_copy(
            inter.at[pl.ds((t * BATCH + base) * EMBED_DIM,
                           SLICE * EMBED_DIM)],
            rows_v.at[pl.ds(t * SLICE * EMBED_DIM, SLICE * EMBED_DIM)],
            sem))
    for c in copies:
        c.wait()

    def chunk_body(c, carry):
        for i in range(LANES):
            r = c * LANES + i
            accp = jnp.zeros((LANES,), jnp.float32)
            accn = jnp.zeros((LANES,), jnp.float32)
            for k in range(EMBED_DIM // LANES):
                o = r * EMBED_DIM + k * LANES
                a = rows_v[pl.ds(o, LANES)]
                p = rows_v[pl.ds(SLICE * EMBED_DIM + o, LANES)]
                n = rows_v[pl.ds(2 * SLICE * EMBED_DIM + o, LANES)]
                accp = accp + a * p
                accn = accn + a * n
            pacc_v[pl.ds(i * LANES, LANES)] = accp
            nacc_v[pl.ds(i * LANES, LANES)] = accn
        totp = jnp.zeros((LANES,), jnp.float32)
        totn = jnp.zeros((LANES,), jnp.float32)
        for d in range(LANES):
            gidx = iota16 * LANES + d
            totp = totp + plsc.load_gather(pacc_v, [gidx])
            totn = totn + plsc.load_gather(nacc_v, [gidx])
        score_pos[pl.ds(c * LANES, LANES)] = totp
        score_neg[pl.ds(c * LANES, LANES)] = totn
        return carry

    lax.fori_loop(0, SLICE // LANES, chunk_body, 0)

    pltpu.sync_copy(score_pos, pos_out.at[pl.ds(base, SLICE)])
    pltpu.sync_copy(score_neg, neg_out.at[pl.ds(base, SLICE)])


def _mesh():
    return plsc.VectorSubcoreMesh(
        core_axis_name="c", subcore_axis_name="s",
        num_cores=NUM_CORES, num_subcores=NUM_SUBCORES)


@jax.jit
def _skipgram_scores(in_tab, out_tab, tail_in, tail_out, idx_all):
    params = pltpu.CompilerParams(
        needs_layout_passes=False, use_tc_tiling_on_sc=True)
    gather = pl.kernel(
        _gather_body,
        out_type=jax.ShapeDtypeStruct((N_INTER,), jnp.float32),
        mesh=_mesh(),
        scratch_types=[
            pltpu.VMEM((ICHUNK,), jnp.int32),
            pltpu.VMEM((EMBED_DIM, BLKW), jnp.float32),
            pltpu.VMEM((EMBED_DIM, BLKW), jnp.float32),
            pltpu.VMEM((2 * EMBED_DIM, EMBED_DIM), jnp.float32),
            pltpu.VMEM((NRING * EMBED_DIM,), jnp.float32),
            pltpu.VMEM((MYCAP,), jnp.int32),
            pltpu.VMEM((MYCAP,), jnp.int32),
            pltpu.SemaphoreType.DMA,
            pltpu.SemaphoreType.DMA,
            pltpu.SemaphoreType.DMA,
        ],
        compiler_params=params,
    )
    inter = gather(in_tab, out_tab, tail_in, tail_out, idx_all)
    score = pl.kernel(
        _score_body,
        out_type=(
            jax.ShapeDtypeStruct((BATCH,), jnp.float32),
            jax.ShapeDtypeStruct((BATCH,), jnp.float32),
        ),
        mesh=_mesh(),
        scratch_types=[
            pltpu.VMEM((3 * SLICE * EMBED_DIM,), jnp.float32),
            pltpu.VMEM((LANES * LANES,), jnp.float32),
            pltpu.VMEM((LANES * LANES,), jnp.float32),
            pltpu.VMEM((SLICE,), jnp.float32),
            pltpu.VMEM((SLICE,), jnp.float32),
            pltpu.SemaphoreType.DMA,
        ],
        compiler_params=params,
    )
    return score(inter)


def kernel(input_labels, pos_labels, neg_labels, in_embed, out_embed):
    in_tab = in_embed.T
    out_tab = out_embed.T
    tail_in = in_embed[VMAIN:, :].T
    tail_out = out_embed[VMAIN:, :].T
    idx_all = jnp.concatenate([
        input_labels.astype(jnp.int32),
        pos_labels.astype(jnp.int32),
        neg_labels.astype(jnp.int32)])
    pos_score, neg_score = _skipgram_scores(
        in_tab, out_tab, tail_in, tail_out, idx_all)
    return pos_score, neg_score.reshape(BATCH, 1)
